# Initial kernel scaffold; baseline (speedup 1.0000x reference)
import jax
import jax.numpy as jnp
from jax.experimental import pallas as pl


def kernel(x):
    raise NotImplementedError("write your pallas kernel here")


# speedup vs baseline: 39.5632x; 39.5632x over previous
import dataclasses
import functools

import jax
import jax.numpy as jnp
from jax import lax
from jax.experimental import pallas as pl
from jax.experimental.pallas import tpu as pltpu
from jax.experimental.pallas import tpu_sc as plsc

ROWS = 128
COLS = 32768
K = 16
L = 16
NC = 2
NS = 16
NA = 4


def _merge_max(a, b):
    return jnp.sort(jnp.maximum(a, jnp.flip(b)))


def _merge_min(a, b):
    return jnp.sort(jnp.minimum(a, jnp.flip(b)))


def kernel(x):
    nw = NC * NS
    rows_per_w = ROWS // nw

    mesh = plsc.VectorSubcoreMesh(core_axis_name="c", subcore_axis_name="s")

    cp = pltpu.CompilerParams()
    if "needs_layout_passes" in pltpu.CompilerParams.__dataclass_fields__:
        cp = dataclasses.replace(cp, needs_layout_passes=False)

    @functools.partial(
        pl.kernel,
        out_type=jax.ShapeDtypeStruct((ROWS, 2 * K), jnp.float32),
        mesh=mesh,
        compiler_params=cp,
        scratch_types=[
            pltpu.VMEM((COLS,), jnp.float32),
            pltpu.VMEM((2 * K,), jnp.float32),
            pltpu.SemaphoreType.DMA,
        ],
    )
    def run(x_hbm, out_hbm, row_v, out_v, sem):
        cid = lax.axis_index("c")
        sid = lax.axis_index("s")
        wid = sid * NC + cid

        @pl.loop(0, rows_per_w)
        def _(r):
            row = wid * rows_per_w + r
            pltpu.async_copy(x_hbm.at[row], row_v, sem).wait()

            neg = jnp.full((L,), -jnp.inf, jnp.float32)
            pos = jnp.full((L,), jnp.inf, jnp.float32)
            init = (neg,) * NA + (pos,) * NA

            def body(i, carry):
                tmaxs = list(carry[:NA])
                tmins = list(carry[NA:])
                for j in range(NA):
                    c = row_v[pl.ds((i * NA + j) * L, L)]
                    cf = jnp.flip(jnp.sort(c))
                    tmaxs[j] = jnp.sort(jnp.maximum(tmaxs[j], cf))
                    tmins[j] = jnp.sort(jnp.minimum(tmins[j], cf))
                return tuple(tmaxs) + tuple(tmins)

            carry = lax.fori_loop(0, COLS // (L * NA), body, init)
            tmaxs, tmins = carry[:NA], carry[NA:]
            tmax = _merge_max(_merge_max(tmaxs[0], tmaxs[1]),
                              _merge_max(tmaxs[2], tmaxs[3]))
            tmin = _merge_min(_merge_min(tmins[0], tmins[1]),
                              _merge_min(tmins[2], tmins[3]))
            out_v[pl.ds(0, K)] = tmin
            out_v[pl.ds(K, K)] = tmax
            pltpu.sync_copy(out_v, out_hbm.at[row])

    return run(x)
